# Initial kernel scaffold; baseline (speedup 1.0000x reference)
import jax
import jax.numpy as jnp
from jax.experimental import pallas as pl


def kernel(x, edge_index, batch, W1, b1, W2, b2, Ws, bs):
    raise NotImplementedError("write your pallas kernel here")


# speedup vs baseline: 11.0607x; 11.0607x over previous
import functools

import jax
import jax.numpy as jnp
from jax import lax
from jax.experimental import pallas as pl
from jax.experimental.pallas import tpu as pltpu
from jax.experimental.pallas import tpu_sc as plsc

NN = 10000
FD = 128
NC = 2
NS = 16
NW = NC * NS
CH = 128
NACC = 10240
RPT = NACC // NS
RB = 2000


def _sc_mesh():
    return plsc.VectorSubcoreMesh(core_axis_name="c", subcore_axis_name="s")


def _sc_degree(dst_pad):
    ept = dst_pad.shape[0] // NW
    nch = ept // CH

    @functools.partial(
        pl.kernel,
        out_type=jax.ShapeDtypeStruct((NC, NACC, FD), jnp.float32),
        mesh=_sc_mesh(),
        scratch_types=[
            pltpu.VMEM((CH,), jnp.int32),
            pltpu.VMEM((CH, FD), jnp.float32),
            pltpu.VMEM_SHARED((NACC, FD), jnp.float32),
        ],
    )
    def k(dst_hbm, out_hbm, idx_v, ones_v, acc_sh):
        c = lax.axis_index("c")
        s = lax.axis_index("s")
        wid = c * NS + s

        def fill(val):
            def fi(i, carry):
                def fj(j, carry2):
                    ones_v[i, pl.ds(j * 16, 16)] = jnp.full((16,), val,
                                                            jnp.float32)
                    return carry2

                return lax.fori_loop(0, FD // 16, fj, carry)

            lax.fori_loop(0, CH, fi, 0)

        fill(0.0)

        def zero_out(j, carry):
            pltpu.sync_copy(ones_v, acc_sh.at[pl.ds(s * RPT + j * CH, CH)])
            return carry

        lax.fori_loop(0, RPT // CH, zero_out, 0)
        fill(1.0)
        plsc.subcore_barrier()

        def body(j, carry):
            base = wid * ept + j * CH
            pltpu.sync_copy(dst_hbm.at[pl.ds(base, CH)], idx_v)
            pltpu.sync_copy(ones_v, acc_sh.at[idx_v], add=True)
            return carry

        lax.fori_loop(0, nch, body, 0)
        plsc.subcore_barrier()

        def copy_out(j, carry):
            pltpu.sync_copy(acc_sh.at[pl.ds(s * RPT + j * CH, CH)], ones_v)
            pltpu.sync_copy(ones_v, out_hbm.at[c, pl.ds(s * RPT + j * CH, CH)])
            return carry

        lax.fori_loop(0, RPT // CH, copy_out, 0)

    return k(dst_pad)


def _sc_aggregate(src_pad, dst_pad, table):
    ept = src_pad.shape[0] // NW
    nch = ept // CH

    @functools.partial(
        pl.kernel,
        out_type=jax.ShapeDtypeStruct((NC, NACC, FD), jnp.float32),
        mesh=_sc_mesh(),
        scratch_types=[
            pltpu.VMEM((CH,), jnp.int32),
            pltpu.VMEM((CH,), jnp.int32),
            pltpu.VMEM((CH, FD), jnp.float32),
            pltpu.VMEM_SHARED((NACC, FD), jnp.float32),
            pltpu.SemaphoreType.DMA,
        ],
    )
    def k(src_hbm, dst_hbm, tab_hbm, out_hbm, sidx_v, didx_v, rows_v,
          acc_sh, sem):
        c = lax.axis_index("c")
        s = lax.axis_index("s")
        wid = c * NS + s

        def fill_zero(i, carry):
            def fz(j, carry2):
                rows_v[i, pl.ds(j * 16, 16)] = jnp.zeros((16,), jnp.float32)
                return carry2

            return lax.fori_loop(0, FD // 16, fz, carry)

        lax.fori_loop(0, CH, fill_zero, 0)

        def zero_out(j, carry):
            pltpu.sync_copy(rows_v, acc_sh.at[pl.ds(s * RPT + j * CH, CH)])
            return carry

        lax.fori_loop(0, RPT // CH, zero_out, 0)
        plsc.subcore_barrier()

        def body(j, carry):
            base = wid * ept + j * CH
            pltpu.sync_copy(src_hbm.at[pl.ds(base, CH)], sidx_v)
            pltpu.sync_copy(dst_hbm.at[pl.ds(base, CH)], didx_v)
            pltpu.async_copy(tab_hbm.at[sidx_v], rows_v, sem).wait()
            pltpu.sync_copy(rows_v, acc_sh.at[didx_v], add=True)
            return carry

        lax.fori_loop(0, nch, body, 0)
        plsc.subcore_barrier()

        def copy_out(j, carry):
            pltpu.sync_copy(acc_sh.at[pl.ds(s * RPT + j * CH, CH)], rows_v)
            pltpu.sync_copy(rows_v, out_hbm.at[c, pl.ds(s * RPT + j * CH, CH)])
            return carry

        lax.fori_loop(0, RPT // CH, copy_out, 0)

    return k(src_pad, dst_pad, table)


def _dis_block(deg_ref):
    deg = deg_ref[0, :, 0:1] + deg_ref[1, :, 0:1] + 1.0
    return lax.rsqrt(deg)


def _tc_layer1(x, w1, degp):
    def body(x_ref, w_ref, deg_ref, h_ref, hs_ref):
        dis = _dis_block(deg_ref)
        h = jnp.dot(x_ref[...], w_ref[...], preferred_element_type=jnp.float32)
        h_ref[...] = h
        hs_ref[...] = h * dis

    return pl.pallas_call(
        body,
        grid=(NN // RB,),
        in_specs=[
            pl.BlockSpec((RB, FD), lambda i: (i, 0)),
            pl.BlockSpec((FD, FD), lambda i: (0, 0)),
            pl.BlockSpec((NC, RB, 8), lambda i: (0, i, 0)),
        ],
        out_specs=[pl.BlockSpec((RB, FD), lambda i: (i, 0))] * 2,
        out_shape=[jax.ShapeDtypeStruct((NN, FD), jnp.float32)] * 2,
    )(x, w1, degp)


def _tc_layer2(aggp, h1, degp, b1, w2):
    def body(agg_ref, h1_ref, deg_ref, b_ref, w_ref, h2_ref, hs2_ref):
        dis = _dis_block(deg_ref)
        agg = agg_ref[0] + agg_ref[1]
        pre = dis * agg + (dis * dis) * h1_ref[...] + b_ref[...]
        h1r = jnp.maximum(pre, 0.0)
        h2 = jnp.dot(h1r, w_ref[...], preferred_element_type=jnp.float32)
        h2_ref[...] = h2
        hs2_ref[...] = h2 * dis

    return pl.pallas_call(
        body,
        grid=(NN // RB,),
        in_specs=[
            pl.BlockSpec((NC, RB, FD), lambda i: (0, i, 0)),
            pl.BlockSpec((RB, FD), lambda i: (i, 0)),
            pl.BlockSpec((NC, RB, 8), lambda i: (0, i, 0)),
            pl.BlockSpec((1, FD), lambda i: (0, 0)),
            pl.BlockSpec((FD, FD), lambda i: (0, 0)),
        ],
        out_specs=[pl.BlockSpec((RB, FD), lambda i: (i, 0))] * 2,
        out_shape=[jax.ShapeDtypeStruct((NN, FD), jnp.float32)] * 2,
    )(aggp, h1, degp, b1, w2)


def _tc_final(aggp, h2, degp, b2, ws, bs):
    nblk = NN // RB

    def body(agg_ref, h2_ref, deg_ref, b_ref, ws_ref, bs_ref, sum_ref,
             hf_ref, acc):
        i = pl.program_id(0)

        @pl.when(i == 0)
        def _init():
            acc[...] = jnp.zeros_like(acc)

        dis = _dis_block(deg_ref)
        agg = agg_ref[0] + agg_ref[1]
        hf = dis * agg + (dis * dis) * h2_ref[...] + b_ref[...]
        hf_ref[...] = hf
        acc[...] += jnp.sum(hf, axis=0, keepdims=True)

        @pl.when(i == nblk - 1)
        def _fin():
            g = acc[...] * jnp.float32(1.0 / NN)
            sum_ref[...] = (
                jnp.dot(g, ws_ref[...], preferred_element_type=jnp.float32)
                + bs_ref[...]
            )

    return pl.pallas_call(
        body,
        grid=(nblk,),
        in_specs=[
            pl.BlockSpec((NC, RB, FD), lambda i: (0, i, 0)),
            pl.BlockSpec((RB, FD), lambda i: (i, 0)),
            pl.BlockSpec((NC, RB, 8), lambda i: (0, i, 0)),
            pl.BlockSpec((1, FD), lambda i: (0, 0)),
            pl.BlockSpec((FD, FD), lambda i: (0, 0)),
            pl.BlockSpec((1, FD), lambda i: (0, 0)),
        ],
        out_specs=[
            pl.BlockSpec((1, FD), lambda i: (0, 0)),
            pl.BlockSpec((RB, FD), lambda i: (i, 0)),
        ],
        out_shape=[
            jax.ShapeDtypeStruct((1, FD), jnp.float32),
            jax.ShapeDtypeStruct((NN, FD), jnp.float32),
        ],
        scratch_shapes=[pltpu.VMEM((1, FD), jnp.float32)],
    )(aggp, h2, degp, b2, ws, bs)


def kernel(x, edge_index, batch, W1, b1, W2, b2, Ws, bs):
    src = edge_index[0]
    dst = edge_index[1]
    e = src.shape[0]
    ept = -(-e // (NW * CH)) * CH
    pad = NW * ept - e
    src_pad = jnp.concatenate([src, jnp.zeros((pad,), jnp.int32)])
    dst_pad = jnp.concatenate([dst, jnp.full((pad,), NN, jnp.int32)])

    degp = _sc_degree(dst_pad)[:, :, :8]
    h1, hs = _tc_layer1(x, W1, degp)
    agg1 = _sc_aggregate(src_pad, dst_pad, hs)
    h2, hs2 = _tc_layer2(agg1, h1, degp, b1.reshape(1, FD), W2)
    agg2 = _sc_aggregate(src_pad, dst_pad, hs2)
    summary, hf = _tc_final(agg2, h2, degp, b2.reshape(1, FD), Ws,
                            bs.reshape(1, FD))
    return (summary, hf)


# speedup vs baseline: 11.2902x; 1.0208x over previous
import functools

import jax
import jax.numpy as jnp
from jax import lax
from jax.experimental import pallas as pl
from jax.experimental.pallas import tpu as pltpu
from jax.experimental.pallas import tpu_sc as plsc

NN = 10000
FD = 128
NC = 2
NS = 16
NW = NC * NS
CH = 128
NACC = 10240
RPT = NACC // NS
RB = 2000


def _sc_mesh():
    return plsc.VectorSubcoreMesh(core_axis_name="c", subcore_axis_name="s")


def _sc_degree(dst_pad):
    ept = dst_pad.shape[0] // NW
    nch = ept // CH

    @functools.partial(
        pl.kernel,
        out_type=jax.ShapeDtypeStruct((NC, NACC, FD), jnp.float32),
        mesh=_sc_mesh(),
        scratch_types=[
            pltpu.VMEM((CH,), jnp.int32),
            pltpu.VMEM((CH, FD), jnp.float32),
            pltpu.VMEM_SHARED((NACC, FD), jnp.float32),
        ],
    )
    def k(dst_hbm, out_hbm, idx_v, ones_v, acc_sh):
        c = lax.axis_index("c")
        s = lax.axis_index("s")
        wid = c * NS + s
        base = wid * ept

        def fill(val):
            def fi(i, carry):
                def fj(j, carry2):
                    ones_v[i, pl.ds(j * 16, 16)] = jnp.full((16,), val,
                                                            jnp.float32)
                    return carry2

                return lax.fori_loop(0, FD // 16, fj, carry)

            lax.fori_loop(0, CH, fi, 0)

        fill(0.0)

        def zero_out(j, carry):
            pltpu.sync_copy(ones_v, acc_sh.at[pl.ds(s * RPT + j * CH, CH)])
            return carry

        lax.fori_loop(0, RPT // CH, zero_out, 0)
        fill(1.0)
        plsc.subcore_barrier()

        def body(j, carry):
            pltpu.sync_copy(dst_hbm.at[pl.ds(base + j * CH, CH)], idx_v)
            pltpu.sync_copy(ones_v, acc_sh.at[idx_v], add=True)
            return carry

        lax.fori_loop(0, nch, body, 0)
        plsc.subcore_barrier()

        def copy_out(j, carry):
            pltpu.sync_copy(acc_sh.at[pl.ds(s * RPT + j * CH, CH)], ones_v)
            pltpu.sync_copy(ones_v, out_hbm.at[c, pl.ds(s * RPT + j * CH, CH)])
            return carry

        lax.fori_loop(0, RPT // CH, copy_out, 0)

    return k(dst_pad)


def _sc_aggregate(eidx, table):
    nch = eidx.shape[0] // NW
    half = nch // 2

    @functools.partial(
        pl.kernel,
        out_type=jax.ShapeDtypeStruct((NC, NACC, FD), jnp.float32),
        mesh=_sc_mesh(),
        scratch_types=[
            pltpu.VMEM((2, CH), jnp.int32),
            pltpu.VMEM((2, CH), jnp.int32),
            pltpu.VMEM((CH,), jnp.int32),
            pltpu.VMEM((CH,), jnp.int32),
            pltpu.VMEM((CH,), jnp.int32),
            pltpu.VMEM((CH,), jnp.int32),
            pltpu.VMEM((CH, FD), jnp.float32),
            pltpu.VMEM((CH, FD), jnp.float32),
            pltpu.VMEM_SHARED((NACC, FD), jnp.float32),
            pltpu.SemaphoreType.DMA,
            pltpu.SemaphoreType.DMA,
            pltpu.SemaphoreType.DMA,
            pltpu.SemaphoreType.DMA,
            pltpu.SemaphoreType.DMA,
            pltpu.SemaphoreType.DMA,
            pltpu.SemaphoreType.DMA,
        ],
    )
    def k(eidx_hbm, tab_hbm, out_hbm, ebufA, ebufB, sidxA, sidxB, didxA,
          didxB, rowsA, rowsB, acc_sh, semEA, semEB, semGA, semGB, semSA,
          semSB, semF):
        c = lax.axis_index("c")
        s = lax.axis_index("s")
        wid = c * NS + s
        cbase = wid * nch

        def fill_zero(i, carry):
            def fz(j, carry2):
                rowsA[i, pl.ds(j * 16, 16)] = jnp.zeros((16,), jnp.float32)
                return carry2

            return lax.fori_loop(0, FD // 16, fz, carry)

        lax.fori_loop(0, CH, fill_zero, 0)

        def zero_out(j, carry):
            pltpu.sync_copy(rowsA, acc_sh.at[pl.ds(s * RPT + j * CH, CH)])
            return carry

        lax.fori_loop(0, RPT // CH, zero_out, 0)
        plsc.subcore_barrier()

        def cpboth(ebuf, sidx, didx):
            def cp(i, carry):
                sidx[pl.ds(i * 16, 16)] = ebuf[0, pl.ds(i * 16, 16)]
                didx[pl.ds(i * 16, 16)] = ebuf[1, pl.ds(i * 16, 16)]
                return carry

            lax.fori_loop(0, CH // 16, cp, 0)

        def body(j, carry):
            cid = cbase + j
            pltpu.sync_copy(eidx_hbm.at[cid], ebufA)
            cpboth(ebufA, sidxA, didxA)
            pltpu.async_copy(tab_hbm.at[sidxA], rowsA, semGA).wait()
            pltpu.async_copy(rowsA, acc_sh.at[didxA], semSA, add=True).wait()
            return carry

        lax.fori_loop(0, nch, body, 0)
        plsc.subcore_barrier()

        def copy_out(j, carry):
            pltpu.sync_copy(acc_sh.at[pl.ds(s * RPT + j * CH, CH)], rowsA)
            pltpu.sync_copy(rowsA, out_hbm.at[c, pl.ds(s * RPT + j * CH, CH)])
            return carry

        lax.fori_loop(0, RPT // CH, copy_out, 0)

    return k(eidx, table)


def _dis_block(deg_ref):
    deg = deg_ref[0, :, 0:1] + deg_ref[1, :, 0:1] + 1.0
    return lax.rsqrt(deg)


def _tc_layer1(x, w1, degp):
    def body(x_ref, w_ref, deg_ref, h_ref, hs_ref):
        dis = _dis_block(deg_ref)
        h = jnp.dot(x_ref[...], w_ref[...], preferred_element_type=jnp.float32)
        h_ref[...] = h
        hs_ref[...] = h * dis

    return pl.pallas_call(
        body,
        grid=(NN // RB,),
        in_specs=[
            pl.BlockSpec((RB, FD), lambda i: (i, 0)),
            pl.BlockSpec((FD, FD), lambda i: (0, 0)),
            pl.BlockSpec((NC, RB, 8), lambda i: (0, i, 0)),
        ],
        out_specs=[pl.BlockSpec((RB, FD), lambda i: (i, 0))] * 2,
        out_shape=[jax.ShapeDtypeStruct((NN, FD), jnp.float32)] * 2,
    )(x, w1, degp)


def _tc_layer2(aggp, h1, degp, b1, w2):
    def body(agg_ref, h1_ref, deg_ref, b_ref, w_ref, h2_ref, hs2_ref):
        dis = _dis_block(deg_ref)
        agg = agg_ref[0] + agg_ref[1]
        pre = dis * agg + (dis * dis) * h1_ref[...] + b_ref[...]
        h1r = jnp.maximum(pre, 0.0)
        h2 = jnp.dot(h1r, w_ref[...], preferred_element_type=jnp.float32)
        h2_ref[...] = h2
        hs2_ref[...] = h2 * dis

    return pl.pallas_call(
        body,
        grid=(NN // RB,),
        in_specs=[
            pl.BlockSpec((NC, RB, FD), lambda i: (0, i, 0)),
            pl.BlockSpec((RB, FD), lambda i: (i, 0)),
            pl.BlockSpec((NC, RB, 8), lambda i: (0, i, 0)),
            pl.BlockSpec((1, FD), lambda i: (0, 0)),
            pl.BlockSpec((FD, FD), lambda i: (0, 0)),
        ],
        out_specs=[pl.BlockSpec((RB, FD), lambda i: (i, 0))] * 2,
        out_shape=[jax.ShapeDtypeStruct((NN, FD), jnp.float32)] * 2,
    )(aggp, h1, degp, b1, w2)


def _tc_final(aggp, h2, degp, b2, ws, bs):
    nblk = NN // RB

    def body(agg_ref, h2_ref, deg_ref, b_ref, ws_ref, bs_ref, sum_ref,
             hf_ref, acc):
        i = pl.program_id(0)

        @pl.when(i == 0)
        def _init():
            acc[...] = jnp.zeros_like(acc)

        dis = _dis_block(deg_ref)
        agg = agg_ref[0] + agg_ref[1]
        hf = dis * agg + (dis * dis) * h2_ref[...] + b_ref[...]
        hf_ref[...] = hf
        acc[...] += jnp.sum(hf, axis=0, keepdims=True)

        @pl.when(i == nblk - 1)
        def _fin():
            g = acc[...] * jnp.float32(1.0 / NN)
            sum_ref[...] = (
                jnp.dot(g, ws_ref[...], preferred_element_type=jnp.float32)
                + bs_ref[...]
            )

    return pl.pallas_call(
        body,
        grid=(nblk,),
        in_specs=[
            pl.BlockSpec((NC, RB, FD), lambda i: (0, i, 0)),
            pl.BlockSpec((RB, FD), lambda i: (i, 0)),
            pl.BlockSpec((NC, RB, 8), lambda i: (0, i, 0)),
            pl.BlockSpec((1, FD), lambda i: (0, 0)),
            pl.BlockSpec((FD, FD), lambda i: (0, 0)),
            pl.BlockSpec((1, FD), lambda i: (0, 0)),
        ],
        out_specs=[
            pl.BlockSpec((1, FD), lambda i: (0, 0)),
            pl.BlockSpec((RB, FD), lambda i: (i, 0)),
        ],
        out_shape=[
            jax.ShapeDtypeStruct((1, FD), jnp.float32),
            jax.ShapeDtypeStruct((NN, FD), jnp.float32),
        ],
        scratch_shapes=[pltpu.VMEM((1, FD), jnp.float32)],
    )(aggp, h2, degp, b2, ws, bs)


def kernel(x, edge_index, batch, W1, b1, W2, b2, Ws, bs):
    src = edge_index[0]
    dst = edge_index[1]
    e = src.shape[0]
    ept = -(-e // (NW * CH)) * CH
    pad = NW * ept - e
    src_pad = jnp.concatenate([src, jnp.zeros((pad,), jnp.int32)])
    dst_pad = jnp.concatenate([dst, jnp.full((pad,), NN, jnp.int32)])
    eidx = jnp.stack(
        [src_pad.reshape(-1, CH), dst_pad.reshape(-1, CH)], axis=1)

    degp = _sc_degree(dst_pad)[:, :, :8]
    h1, hs = _tc_layer1(x, W1, degp)
    agg1 = _sc_aggregate(eidx, hs)
    h2, hs2 = _tc_layer2(agg1, h1, degp, b1.reshape(1, FD), W2)
    agg2 = _sc_aggregate(eidx, hs2)
    summary, hf = _tc_final(agg2, h2, degp, b2.reshape(1, FD), Ws,
                            bs.reshape(1, FD))
    return (summary, hf)


# speedup vs baseline: 11.2938x; 1.0003x over previous
import functools

import jax
import jax.numpy as jnp
from jax import lax
from jax.experimental import pallas as pl
from jax.experimental.pallas import tpu as pltpu
from jax.experimental.pallas import tpu_sc as plsc

NN = 10000
FD = 128
NC = 2
NS = 16
NW = NC * NS
CH = 128
NACC = 10240
RPT = NACC // NS
RB = 2000


def _sc_mesh():
    return plsc.VectorSubcoreMesh(core_axis_name="c", subcore_axis_name="s")


def _sc_degree(dst_pad):
    ept = dst_pad.shape[0] // NW
    nch = ept // CH

    @functools.partial(
        pl.kernel,
        out_type=jax.ShapeDtypeStruct((NC, NACC, FD), jnp.float32),
        mesh=_sc_mesh(),
        scratch_types=[
            pltpu.VMEM((CH,), jnp.int32),
            pltpu.VMEM((CH, FD), jnp.float32),
            pltpu.VMEM_SHARED((NACC, FD), jnp.float32),
        ],
    )
    def k(dst_hbm, out_hbm, idx_v, ones_v, acc_sh):
        c = lax.axis_index("c")
        s = lax.axis_index("s")
        wid = c * NS + s
        base = wid * ept

        def fill(val):
            def fi(i, carry):
                def fj(j, carry2):
                    ones_v[i, pl.ds(j * 16, 16)] = jnp.full((16,), val,
                                                            jnp.float32)
                    return carry2

                return lax.fori_loop(0, FD // 16, fj, carry)

            lax.fori_loop(0, CH, fi, 0)

        fill(0.0)

        def zero_out(j, carry):
            pltpu.sync_copy(ones_v, acc_sh.at[pl.ds(s * RPT + j * CH, CH)])
            return carry

        lax.fori_loop(0, RPT // CH, zero_out, 0)
        fill(1.0)
        plsc.subcore_barrier()

        def body(j, carry):
            pltpu.sync_copy(dst_hbm.at[pl.ds(base + j * CH, CH)], idx_v)
            pltpu.sync_copy(ones_v, acc_sh.at[idx_v], add=True)
            return carry

        lax.fori_loop(0, nch, body, 0)
        plsc.subcore_barrier()

        def copy_out(j, carry):
            pltpu.sync_copy(acc_sh.at[pl.ds(s * RPT + j * CH, CH)], ones_v)
            pltpu.sync_copy(ones_v, out_hbm.at[c, pl.ds(s * RPT + j * CH, CH)])
            return carry

        lax.fori_loop(0, RPT // CH, copy_out, 0)

    return k(dst_pad)


def _sc_aggregate(eidx, table):
    nch = eidx.shape[0] // NW

    @functools.partial(
        pl.kernel,
        out_type=jax.ShapeDtypeStruct((NC, NACC, FD), jnp.float32),
        mesh=_sc_mesh(),
        scratch_types=[
            pltpu.VMEM((2, CH), jnp.int32),
            pltpu.VMEM((CH,), jnp.int32),
            pltpu.VMEM((CH,), jnp.int32),
            pltpu.VMEM((CH, FD), jnp.float32),
            pltpu.VMEM_SHARED((NACC, FD), jnp.float32),
            pltpu.SemaphoreType.DMA,
            pltpu.SemaphoreType.DMA,
        ],
    )
    def k(eidx_hbm, tab_hbm, out_hbm, ebufA, sidxA, didxA, rowsA, acc_sh,
          semGA, semSA):
        c = lax.axis_index("c")
        s = lax.axis_index("s")
        wid = c * NS + s
        cbase = wid * nch

        def fill_zero(i, carry):
            def fz(j, carry2):
                rowsA[i, pl.ds(j * 16, 16)] = jnp.zeros((16,), jnp.float32)
                return carry2

            return lax.fori_loop(0, FD // 16, fz, carry)

        lax.fori_loop(0, CH, fill_zero, 0)

        def zero_out(j, carry):
            pltpu.sync_copy(rowsA, acc_sh.at[pl.ds(s * RPT + j * CH, CH)])
            return carry

        lax.fori_loop(0, RPT // CH, zero_out, 0)
        plsc.subcore_barrier()

        def cpboth(ebuf, sidx, didx):
            def cp(i, carry):
                sidx[pl.ds(i * 16, 16)] = ebuf[0, pl.ds(i * 16, 16)]
                didx[pl.ds(i * 16, 16)] = ebuf[1, pl.ds(i * 16, 16)]
                return carry

            lax.fori_loop(0, CH // 16, cp, 0)

        def body(j, carry):
            cid = cbase + j
            pltpu.sync_copy(eidx_hbm.at[cid], ebufA)
            cpboth(ebufA, sidxA, didxA)
            pltpu.async_copy(tab_hbm.at[sidxA], rowsA, semGA).wait()
            pltpu.async_copy(rowsA, acc_sh.at[didxA], semSA, add=True).wait()
            return carry

        lax.fori_loop(0, nch, body, 0)
        plsc.subcore_barrier()

        def copy_out(j, carry):
            pltpu.sync_copy(acc_sh.at[pl.ds(s * RPT + j * CH, CH)], rowsA)
            pltpu.sync_copy(rowsA, out_hbm.at[c, pl.ds(s * RPT + j * CH, CH)])
            return carry

        lax.fori_loop(0, RPT // CH, copy_out, 0)

    return k(eidx, table)


def _dis_block(deg_ref):
    deg = deg_ref[0, :, 0:1] + deg_ref[1, :, 0:1] + 1.0
    return lax.rsqrt(deg)


def _tc_layer1(x, w1, degp):
    def body(x_ref, w_ref, deg_ref, h_ref, hs_ref):
        dis = _dis_block(deg_ref)
        h = jnp.dot(x_ref[...], w_ref[...], preferred_element_type=jnp.float32)
        h_ref[...] = h
        hs_ref[...] = h * dis

    return pl.pallas_call(
        body,
        grid=(NN // RB,),
        in_specs=[
            pl.BlockSpec((RB, FD), lambda i: (i, 0)),
            pl.BlockSpec((FD, FD), lambda i: (0, 0)),
            pl.BlockSpec((NC, RB, 8), lambda i: (0, i, 0)),
        ],
        out_specs=[pl.BlockSpec((RB, FD), lambda i: (i, 0))] * 2,
        out_shape=[jax.ShapeDtypeStruct((NN, FD), jnp.float32)] * 2,
    )(x, w1, degp)


def _tc_layer2(aggp, h1, degp, b1, w2):
    def body(agg_ref, h1_ref, deg_ref, b_ref, w_ref, h2_ref, hs2_ref):
        dis = _dis_block(deg_ref)
        agg = agg_ref[0] + agg_ref[1]
        pre = dis * agg + (dis * dis) * h1_ref[...] + b_ref[...]
        h1r = jnp.maximum(pre, 0.0)
        h2 = jnp.dot(h1r, w_ref[...], preferred_element_type=jnp.float32)
        h2_ref[...] = h2
        hs2_ref[...] = h2 * dis

    return pl.pallas_call(
        body,
        grid=(NN // RB,),
        in_specs=[
            pl.BlockSpec((NC, RB, FD), lambda i: (0, i, 0)),
            pl.BlockSpec((RB, FD), lambda i: (i, 0)),
            pl.BlockSpec((NC, RB, 8), lambda i: (0, i, 0)),
            pl.BlockSpec((1, FD), lambda i: (0, 0)),
            pl.BlockSpec((FD, FD), lambda i: (0, 0)),
        ],
        out_specs=[pl.BlockSpec((RB, FD), lambda i: (i, 0))] * 2,
        out_shape=[jax.ShapeDtypeStruct((NN, FD), jnp.float32)] * 2,
    )(aggp, h1, degp, b1, w2)


def _tc_final(aggp, h2, degp, b2, ws, bs):
    nblk = NN // RB

    def body(agg_ref, h2_ref, deg_ref, b_ref, ws_ref, bs_ref, sum_ref,
             hf_ref, acc):
        i = pl.program_id(0)

        @pl.when(i == 0)
        def _init():
            acc[...] = jnp.zeros_like(acc)

        dis = _dis_block(deg_ref)
        agg = agg_ref[0] + agg_ref[1]
        hf = dis * agg + (dis * dis) * h2_ref[...] + b_ref[...]
        hf_ref[...] = hf
        acc[...] += jnp.sum(hf, axis=0, keepdims=True)

        @pl.when(i == nblk - 1)
        def _fin():
            g = acc[...] * jnp.float32(1.0 / NN)
            sum_ref[...] = (
                jnp.dot(g, ws_ref[...], preferred_element_type=jnp.float32)
                + bs_ref[...]
            )

    return pl.pallas_call(
        body,
        grid=(nblk,),
        in_specs=[
            pl.BlockSpec((NC, RB, FD), lambda i: (0, i, 0)),
            pl.BlockSpec((RB, FD), lambda i: (i, 0)),
            pl.BlockSpec((NC, RB, 8), lambda i: (0, i, 0)),
            pl.BlockSpec((1, FD), lambda i: (0, 0)),
            pl.BlockSpec((FD, FD), lambda i: (0, 0)),
            pl.BlockSpec((1, FD), lambda i: (0, 0)),
        ],
        out_specs=[
            pl.BlockSpec((1, FD), lambda i: (0, 0)),
            pl.BlockSpec((RB, FD), lambda i: (i, 0)),
        ],
        out_shape=[
            jax.ShapeDtypeStruct((1, FD), jnp.float32),
            jax.ShapeDtypeStruct((NN, FD), jnp.float32),
        ],
        scratch_shapes=[pltpu.VMEM((1, FD), jnp.float32)],
    )(aggp, h2, degp, b2, ws, bs)


def kernel(x, edge_index, batch, W1, b1, W2, b2, Ws, bs):
    src = edge_index[0]
    dst = edge_index[1]
    e = src.shape[0]
    ept = -(-e // (NW * CH)) * CH
    pad = NW * ept - e
    src_pad = jnp.concatenate([src, jnp.zeros((pad,), jnp.int32)])
    dst_pad = jnp.concatenate([dst, jnp.full((pad,), NN, jnp.int32)])
    eidx = jnp.stack(
        [src_pad.reshape(-1, CH), dst_pad.reshape(-1, CH)], axis=1)

    degp = _sc_degree(dst_pad)[:, :, :8]
    h1, hs = _tc_layer1(x, W1, degp)
    agg1 = _sc_aggregate(eidx, hs)
    h2, hs2 = _tc_layer2(agg1, h1, degp, b1.reshape(1, FD), W2)
    agg2 = _sc_aggregate(eidx, hs2)
    summary, hf = _tc_final(agg2, h2, degp, b2.reshape(1, FD), Ws,
                            bs.reshape(1, FD))
    return (summary, hf)
